# baseline (speedup 1.0000x reference)
import jax
import jax.numpy as jnp
from jax import lax
from jax.experimental import pallas as pl
from jax.experimental.pallas import tpu as pltpu
from jax.experimental.pallas import tpu_sc as plsc

N_NODES = 50000
N_EDGES = 1600000
NC = 2
NS = 16
NW = NC * NS
LANES = 16

EDGES_PER_TILE = N_EDGES // NW
CHUNK = 2000
NCHUNKS = EDGES_PER_TILE // CHUNK
GATHER = 80
NGATHER = CHUNK // GATHER
NGROUPS = CHUNK // LANES

_MAGIC = 0x5F3759DF


def _tec_body(px_hbm, py_hbm, pz_hbm, nli_hbm, nlj_hbm,
              shx_hbm, shy_hbm, shz_hbm, par_hbm, out_hbm,
              px_sh, py_sh, pz_sh, idx_i, idx_j, shxv, shyv, shzv,
              xib, yib, zib, xjb, yjb, zjb,
              parv, accv, sem_i, sem_j):
    cid = lax.axis_index("c")
    sid = lax.axis_index("s")
    wid = sid * NC + cid

    @pl.when(sid == 0)
    def _stage():
        pltpu.sync_copy(px_hbm, px_sh)
        pltpu.sync_copy(py_hbm, py_sh)
        pltpu.sync_copy(pz_hbm, pz_sh)

    pltpu.sync_copy(par_hbm, parv)
    plsc.subcore_barrier()

    a1 = parv[0, :]
    a2 = parv[1, :]
    b1 = parv[2, :]
    b2 = parv[3, :]
    ev = parv[4, :]

    half = jnp.full((LANES,), 0.5, jnp.float32)
    three_half = jnp.full((LANES,), 1.5, jnp.float32)
    one = jnp.full((LANES,), 1.0, jnp.float32)
    zero = jnp.zeros((LANES,), jnp.float32)
    magic = jnp.full((LANES,), _MAGIC, jnp.int32)

    tile_base = wid * EDGES_PER_TILE

    def chunk_body(ci, acc):
        ebase = tile_base + ci * CHUNK
        sl = pl.ds(ebase, CHUNK)
        pltpu.sync_copy(nli_hbm.at[sl], idx_i)
        pltpu.sync_copy(nlj_hbm.at[sl], idx_j)
        pltpu.sync_copy(shx_hbm.at[sl], shxv)
        pltpu.sync_copy(shy_hbm.at[sl], shyv)
        pltpu.sync_copy(shz_hbm.at[sl], shzv)

        descs = []
        for k in range(NGATHER):
            gsl = pl.ds(k * GATHER, GATHER)
            ii = idx_i.at[gsl]
            jj = idx_j.at[gsl]
            descs.append(pltpu.async_copy(px_sh.at[ii], xib.at[gsl], sem_i))
            descs.append(pltpu.async_copy(py_sh.at[ii], yib.at[gsl], sem_i))
            descs.append(pltpu.async_copy(pz_sh.at[ii], zib.at[gsl], sem_i))
            descs.append(pltpu.async_copy(px_sh.at[jj], xjb.at[gsl], sem_j))
            descs.append(pltpu.async_copy(py_sh.at[jj], yjb.at[gsl], sem_j))
            descs.append(pltpu.async_copy(pz_sh.at[jj], zjb.at[gsl], sem_j))
        for d in descs:
            d.wait()

        def group_body(g, acc_in):
            base = g * LANES
            sl16 = pl.ds(base, LANES)
            dx = xjb[sl16] - xib[sl16] + shxv[sl16]
            dy = yjb[sl16] - yib[sl16] + shyv[sl16]
            dz = zjb[sl16] - zib[sl16] + shzv[sl16]
            d2 = dx * dx + dy * dy + dz * dz
            d2 = jnp.maximum(d2, jnp.full((LANES,), 1e-30, jnp.float32))
            y = plsc.bitcast(magic - (plsc.bitcast(d2, jnp.int32) >> 1),
                             jnp.float32)
            xh = half * d2
            y = y * (three_half - xh * y * y)
            y = y * (three_half - xh * y * y)
            y = y * (three_half - xh * y * y)
            dist = d2 * y
            expf = jnp.exp(a1 - a2 * dist)
            s = b1 - b2 * dist
            s3 = (s * s) * s
            poly = ((jnp.full((LANES,), 6.0, jnp.float32) * s
                     - jnp.full((LANES,), 15.0, jnp.float32)) * s
                    + jnp.full((LANES,), 10.0, jnp.float32)) * s3
            fc = jnp.where(s >= one, one, jnp.maximum(poly, zero))
            return acc_in + expf * (expf - jnp.full((LANES,), 2.0,
                                                    jnp.float32)) * fc

        return lax.fori_loop(0, NGROUPS, group_body, acc)

    acc = lax.fori_loop(0, NCHUNKS, chunk_body,
                        jnp.zeros((LANES,), jnp.float32))

    accv[...] = acc * ev
    pltpu.sync_copy(accv, out_hbm.at[wid])


@jax.jit
def _run(px, py, pz, nli, nlj, shx, shy, shz, params):
    mesh = plsc.VectorSubcoreMesh(core_axis_name="c", subcore_axis_name="s")
    kfn = pl.kernel(
        _tec_body,
        out_type=jax.ShapeDtypeStruct((NW, LANES), jnp.float32),
        mesh=mesh,
        scratch_types=[
            pltpu.MemorySpace.VMEM_SHARED((N_NODES,), jnp.float32),
            pltpu.MemorySpace.VMEM_SHARED((N_NODES,), jnp.float32),
            pltpu.MemorySpace.VMEM_SHARED((N_NODES,), jnp.float32),
            pltpu.MemorySpace.VMEM((CHUNK,), jnp.int32),
            pltpu.MemorySpace.VMEM((CHUNK,), jnp.int32),
            pltpu.MemorySpace.VMEM((CHUNK,), jnp.float32),
            pltpu.MemorySpace.VMEM((CHUNK,), jnp.float32),
            pltpu.MemorySpace.VMEM((CHUNK,), jnp.float32),
            pltpu.MemorySpace.VMEM((CHUNK,), jnp.float32),
            pltpu.MemorySpace.VMEM((CHUNK,), jnp.float32),
            pltpu.MemorySpace.VMEM((CHUNK,), jnp.float32),
            pltpu.MemorySpace.VMEM((CHUNK,), jnp.float32),
            pltpu.MemorySpace.VMEM((CHUNK,), jnp.float32),
            pltpu.MemorySpace.VMEM((CHUNK,), jnp.float32),
            pltpu.MemorySpace.VMEM((8, LANES), jnp.float32),
            pltpu.MemorySpace.VMEM((LANES,), jnp.float32),
            pltpu.SemaphoreType.DMA,
            pltpu.SemaphoreType.DMA,
        ],
        compiler_params=pltpu.CompilerParams(needs_layout_passes=False),
    )
    return kfn(px, py, pz, nli, nlj, shx, shy, shz, params)


def kernel(positions, neigh_list, shifts, alpha, epsilon, r0, rcut1, rcut2):
    px = positions[:, 0]
    py = positions[:, 1]
    pz = positions[:, 2]
    nli = neigh_list[0]
    nlj = neigh_list[1]
    shx = shifts[:, 0]
    shy = shifts[:, 1]
    shz = shifts[:, 2]
    inv = 1.0 / (rcut2 - rcut1)
    rows = [
        jnp.broadcast_to(alpha, (LANES,)),
        jnp.broadcast_to(alpha / r0, (LANES,)),
        jnp.broadcast_to(1.0 + rcut1 * inv, (LANES,)),
        jnp.broadcast_to(inv, (LANES,)),
        jnp.broadcast_to(0.5 * epsilon, (LANES,)),
        jnp.zeros((LANES,), jnp.float32),
        jnp.zeros((LANES,), jnp.float32),
        jnp.zeros((LANES,), jnp.float32),
    ]
    params = jnp.stack(rows).astype(jnp.float32)
    out = _run(px, py, pz, nli, nlj, shx, shy, shz, params)
    energy = jnp.sum(out)
    return (energy,)


# speedup vs baseline: 1.1828x; 1.1828x over previous
import jax
import jax.numpy as jnp
from jax import lax
from jax.experimental import pallas as pl
from jax.experimental.pallas import tpu as pltpu
from jax.experimental.pallas import tpu_sc as plsc

N_NODES = 50000
N_EDGES = 1600000
NC = 2
NS = 16
NW = NC * NS
LANES = 16

EDGES_PER_TILE = N_EDGES // NW
CHUNK = 2000
NCHUNKS = EDGES_PER_TILE // CHUNK
GATHER = 80
NGATHER = CHUNK // GATHER
NGROUPS = CHUNK // LANES

_MAGIC = 0x5F3759DF


def _tec_body(px_hbm, py_hbm, pz_hbm, nli_hbm, nlj_hbm,
              shx_hbm, shy_hbm, shz_hbm, par_hbm, out_hbm,
              px_sh, py_sh, pz_sh, idx_i, idx_j, shxv, shyv, shzv,
              xib, yib, zib, xjb, yjb, zjb,
              parv, accv, sem_i, sem_j, sem_x, sem_s):
    cid = lax.axis_index("c")
    sid = lax.axis_index("s")
    wid = sid * NC + cid

    @pl.when(sid == 0)
    def _stage():
        pltpu.sync_copy(px_hbm, px_sh)
        pltpu.sync_copy(py_hbm, py_sh)
        pltpu.sync_copy(pz_hbm, pz_sh)

    pltpu.sync_copy(par_hbm, parv)
    plsc.subcore_barrier()

    a1 = parv[0, :]
    a2 = parv[1, :]
    b1 = parv[2, :]
    b2 = parv[3, :]
    ev = parv[4, :]

    half = jnp.full((LANES,), 0.5, jnp.float32)
    three_half = jnp.full((LANES,), 1.5, jnp.float32)
    one = jnp.full((LANES,), 1.0, jnp.float32)
    zero = jnp.zeros((LANES,), jnp.float32)
    magic = jnp.full((LANES,), _MAGIC, jnp.int32)

    tile_base = wid * EDGES_PER_TILE

    def chunk_body(ci, acc):
        ebase = tile_base + ci * CHUNK
        sl = pl.ds(ebase, CHUNK)
        lin = [pltpu.async_copy(nli_hbm.at[sl], idx_i, sem_x),
               pltpu.async_copy(nlj_hbm.at[sl], idx_j, sem_x),
               pltpu.async_copy(shx_hbm.at[sl], shxv, sem_s),
               pltpu.async_copy(shy_hbm.at[sl], shyv, sem_s),
               pltpu.async_copy(shz_hbm.at[sl], shzv, sem_s)]
        lin[0].wait()
        lin[1].wait()

        descs = [lin[2], lin[3], lin[4]]
        for k in range(NGATHER):
            gsl = pl.ds(k * GATHER, GATHER)
            ii = idx_i.at[gsl]
            jj = idx_j.at[gsl]
            descs.append(pltpu.async_copy(px_sh.at[ii], xib.at[gsl], sem_i))
            descs.append(pltpu.async_copy(py_sh.at[ii], yib.at[gsl], sem_i))
            descs.append(pltpu.async_copy(pz_sh.at[ii], zib.at[gsl], sem_i))
            descs.append(pltpu.async_copy(px_sh.at[jj], xjb.at[gsl], sem_j))
            descs.append(pltpu.async_copy(py_sh.at[jj], yjb.at[gsl], sem_j))
            descs.append(pltpu.async_copy(pz_sh.at[jj], zjb.at[gsl], sem_j))
        for d in descs:
            d.wait()

        def group_body(g, acc_in):
            base = g * LANES
            sl16 = pl.ds(base, LANES)
            dx = xjb[sl16] - xib[sl16] + shxv[sl16]
            dy = yjb[sl16] - yib[sl16] + shyv[sl16]
            dz = zjb[sl16] - zib[sl16] + shzv[sl16]
            d2 = dx * dx + dy * dy + dz * dz
            d2 = jnp.maximum(d2, jnp.full((LANES,), 1e-30, jnp.float32))
            y = plsc.bitcast(magic - (plsc.bitcast(d2, jnp.int32) >> 1),
                             jnp.float32)
            xh = half * d2
            y = y * (three_half - xh * y * y)
            y = y * (three_half - xh * y * y)
            y = y * (three_half - xh * y * y)
            dist = d2 * y
            expf = jnp.exp(a1 - a2 * dist)
            s = b1 - b2 * dist
            s3 = (s * s) * s
            poly = ((jnp.full((LANES,), 6.0, jnp.float32) * s
                     - jnp.full((LANES,), 15.0, jnp.float32)) * s
                    + jnp.full((LANES,), 10.0, jnp.float32)) * s3
            fc = jnp.where(s >= one, one, jnp.maximum(poly, zero))
            return acc_in + expf * (expf - jnp.full((LANES,), 2.0,
                                                    jnp.float32)) * fc

        return lax.fori_loop(0, NGROUPS, group_body, acc)

    acc = lax.fori_loop(0, NCHUNKS, chunk_body,
                        jnp.zeros((LANES,), jnp.float32))

    accv[...] = acc * ev
    pltpu.sync_copy(accv, out_hbm.at[wid])


@jax.jit
def _run(px, py, pz, nli, nlj, shx, shy, shz, params):
    mesh = plsc.VectorSubcoreMesh(core_axis_name="c", subcore_axis_name="s")
    kfn = pl.kernel(
        _tec_body,
        out_type=jax.ShapeDtypeStruct((NW, LANES), jnp.float32),
        mesh=mesh,
        scratch_types=[
            pltpu.MemorySpace.VMEM_SHARED((N_NODES,), jnp.float32),
            pltpu.MemorySpace.VMEM_SHARED((N_NODES,), jnp.float32),
            pltpu.MemorySpace.VMEM_SHARED((N_NODES,), jnp.float32),
            pltpu.MemorySpace.VMEM((CHUNK,), jnp.int32),
            pltpu.MemorySpace.VMEM((CHUNK,), jnp.int32),
            pltpu.MemorySpace.VMEM((CHUNK,), jnp.float32),
            pltpu.MemorySpace.VMEM((CHUNK,), jnp.float32),
            pltpu.MemorySpace.VMEM((CHUNK,), jnp.float32),
            pltpu.MemorySpace.VMEM((CHUNK,), jnp.float32),
            pltpu.MemorySpace.VMEM((CHUNK,), jnp.float32),
            pltpu.MemorySpace.VMEM((CHUNK,), jnp.float32),
            pltpu.MemorySpace.VMEM((CHUNK,), jnp.float32),
            pltpu.MemorySpace.VMEM((CHUNK,), jnp.float32),
            pltpu.MemorySpace.VMEM((CHUNK,), jnp.float32),
            pltpu.MemorySpace.VMEM((8, LANES), jnp.float32),
            pltpu.MemorySpace.VMEM((LANES,), jnp.float32),
            pltpu.SemaphoreType.DMA,
            pltpu.SemaphoreType.DMA,
            pltpu.SemaphoreType.DMA,
            pltpu.SemaphoreType.DMA,
        ],
        compiler_params=pltpu.CompilerParams(needs_layout_passes=False),
    )
    return kfn(px, py, pz, nli, nlj, shx, shy, shz, params)


def kernel(positions, neigh_list, shifts, alpha, epsilon, r0, rcut1, rcut2):
    px = positions[:, 0]
    py = positions[:, 1]
    pz = positions[:, 2]
    nli = neigh_list[0]
    nlj = neigh_list[1]
    shx = shifts[:, 0]
    shy = shifts[:, 1]
    shz = shifts[:, 2]
    inv = 1.0 / (rcut2 - rcut1)
    rows = [
        jnp.broadcast_to(alpha, (LANES,)),
        jnp.broadcast_to(alpha / r0, (LANES,)),
        jnp.broadcast_to(1.0 + rcut1 * inv, (LANES,)),
        jnp.broadcast_to(inv, (LANES,)),
        jnp.broadcast_to(0.5 * epsilon, (LANES,)),
        jnp.zeros((LANES,), jnp.float32),
        jnp.zeros((LANES,), jnp.float32),
        jnp.zeros((LANES,), jnp.float32),
    ]
    params = jnp.stack(rows).astype(jnp.float32)
    out = _run(px, py, pz, nli, nlj, shx, shy, shz, params)
    energy = jnp.sum(out)
    return (energy,)


# speedup vs baseline: 1.4697x; 1.2425x over previous
import jax
import jax.numpy as jnp
from jax import lax
from jax.experimental import pallas as pl
from jax.experimental.pallas import tpu as pltpu
from jax.experimental.pallas import tpu_sc as plsc

N_NODES = 50000
N_EDGES = 1600000
NC = 2
NS = 16
NW = NC * NS
LANES = 16

BLK = 128
NBLK = N_EDGES // BLK
BLK_MAIN = NBLK // NW
NEXTRA = NBLK - BLK_MAIN * NW
CPB = 16
CHUNK = CPB * BLK
NCHUNKS = BLK_MAIN // CPB
TAIL0 = NCHUNKS * CPB
NGROUPS = CHUNK // LANES

_MAGIC = 0x5F3759DF


def _tec_body(px_hbm, py_hbm, pz_hbm, nl_hbm,
              shx_hbm, shy_hbm, shz_hbm, par_hbm, out_hbm,
              px_sh, py_sh, pz_sh, nlv, shxv, shyv, shzv,
              xib, yib, zib, xjb, yjb, zjb,
              parv, accv, sem_i, sem_j, sem_x, sem_s):
    cid = lax.axis_index("c")
    sid = lax.axis_index("s")
    wid = sid * NC + cid

    @pl.when(sid == 0)
    def _stage():
        pltpu.sync_copy(px_hbm, px_sh)
        pltpu.sync_copy(py_hbm, py_sh)
        pltpu.sync_copy(pz_hbm, pz_sh)

    pltpu.sync_copy(par_hbm, parv)
    plsc.subcore_barrier()

    a1 = parv[0, :]
    a2 = parv[1, :]
    b1 = parv[2, :]
    b2 = parv[3, :]
    ev = parv[4, :]

    half = jnp.full((LANES,), 0.5, jnp.float32)
    three_half = jnp.full((LANES,), 1.5, jnp.float32)
    one = jnp.full((LANES,), 1.0, jnp.float32)
    zero = jnp.zeros((LANES,), jnp.float32)
    magic = jnp.full((LANES,), _MAGIC, jnp.int32)

    base_blk = wid * BLK_MAIN + jnp.minimum(wid, NEXTRA)
    n_tail = BLK_MAIN - TAIL0 + jnp.where(wid < NEXTRA, 1, 0)

    def morse16(dx, dy, dz, acc_in):
        d2 = dx * dx + dy * dy + dz * dz
        d2 = jnp.maximum(d2, jnp.full((LANES,), 1e-30, jnp.float32))
        y = plsc.bitcast(magic - (plsc.bitcast(d2, jnp.int32) >> 1),
                         jnp.float32)
        xh = half * d2
        y = y * (three_half - xh * y * y)
        y = y * (three_half - xh * y * y)
        y = y * (three_half - xh * y * y)
        dist = d2 * y
        expf = jnp.exp(a1 - a2 * dist)
        s = b1 - b2 * dist
        s3 = (s * s) * s
        poly = ((jnp.full((LANES,), 6.0, jnp.float32) * s
                 - jnp.full((LANES,), 15.0, jnp.float32)) * s
                + jnp.full((LANES,), 10.0, jnp.float32)) * s3
        fc = jnp.where(s >= one, one, jnp.maximum(poly, zero))
        return acc_in + expf * (expf - jnp.full((LANES,), 2.0,
                                                jnp.float32)) * fc

    def compute(nedges, acc):
        def group_body(g, acc_in):
            base = g * LANES
            sl16 = pl.ds(base, LANES)
            dx = xjb[sl16] - xib[sl16] + shxv[sl16]
            dy = yjb[sl16] - yib[sl16] + shyv[sl16]
            dz = zjb[sl16] - zib[sl16] + shzv[sl16]
            return morse16(dx, dy, dz, acc_in)
        return lax.fori_loop(0, nedges // LANES, group_body, acc)

    def chunk_body(ci, acc):
        eb = pl.multiple_of((base_blk + ci * CPB) * BLK, CHUNK // CPB * 8)
        sl = pl.ds(eb, CHUNK)
        lin = [pltpu.async_copy(nl_hbm.at[:, sl], nlv, sem_x),
               pltpu.async_copy(shx_hbm.at[sl], shxv, sem_s),
               pltpu.async_copy(shy_hbm.at[sl], shyv, sem_s),
               pltpu.async_copy(shz_hbm.at[sl], shzv, sem_s)]
        lin[0].wait()

        descs = [lin[1], lin[2], lin[3]]
        for k in range(CPB):
            gsl = pl.ds(k * BLK, BLK)
            ii = nlv.at[0, gsl]
            jj = nlv.at[1, gsl]
            descs.append(pltpu.async_copy(px_sh.at[ii], xib.at[gsl], sem_i))
            descs.append(pltpu.async_copy(py_sh.at[ii], yib.at[gsl], sem_i))
            descs.append(pltpu.async_copy(pz_sh.at[ii], zib.at[gsl], sem_i))
            descs.append(pltpu.async_copy(px_sh.at[jj], xjb.at[gsl], sem_j))
            descs.append(pltpu.async_copy(py_sh.at[jj], yjb.at[gsl], sem_j))
            descs.append(pltpu.async_copy(pz_sh.at[jj], zjb.at[gsl], sem_j))
        for d in descs:
            d.wait()

        return compute(CHUNK, acc)

    acc = lax.fori_loop(0, NCHUNKS, chunk_body,
                        jnp.zeros((LANES,), jnp.float32))

    def tail_body(t, acc):
        eb = pl.multiple_of((base_blk + TAIL0 + t) * BLK, BLK)
        sl = pl.ds(eb, BLK)
        slv = pl.ds(0, BLK)
        lin = [pltpu.async_copy(nl_hbm.at[:, sl], nlv.at[:, slv], sem_x),
               pltpu.async_copy(shx_hbm.at[sl], shxv.at[slv], sem_s),
               pltpu.async_copy(shy_hbm.at[sl], shyv.at[slv], sem_s),
               pltpu.async_copy(shz_hbm.at[sl], shzv.at[slv], sem_s)]
        lin[0].wait()
        ii = nlv.at[0, slv]
        jj = nlv.at[1, slv]
        descs = [lin[1], lin[2], lin[3],
                 pltpu.async_copy(px_sh.at[ii], xib.at[slv], sem_i),
                 pltpu.async_copy(py_sh.at[ii], yib.at[slv], sem_i),
                 pltpu.async_copy(pz_sh.at[ii], zib.at[slv], sem_i),
                 pltpu.async_copy(px_sh.at[jj], xjb.at[slv], sem_j),
                 pltpu.async_copy(py_sh.at[jj], yjb.at[slv], sem_j),
                 pltpu.async_copy(pz_sh.at[jj], zjb.at[slv], sem_j)]
        for d in descs:
            d.wait()
        return compute(BLK, acc)

    acc = lax.fori_loop(0, n_tail, tail_body, acc)

    accv[...] = acc * ev
    pltpu.sync_copy(accv, out_hbm.at[wid])


@jax.jit
def _run(px, py, pz, nl, shx, shy, shz, params):
    mesh = plsc.VectorSubcoreMesh(core_axis_name="c", subcore_axis_name="s")
    kfn = pl.kernel(
        _tec_body,
        out_type=jax.ShapeDtypeStruct((NW, LANES), jnp.float32),
        mesh=mesh,
        scratch_types=[
            pltpu.MemorySpace.VMEM_SHARED((N_NODES,), jnp.float32),
            pltpu.MemorySpace.VMEM_SHARED((N_NODES,), jnp.float32),
            pltpu.MemorySpace.VMEM_SHARED((N_NODES,), jnp.float32),
            pltpu.MemorySpace.VMEM((2, CHUNK), jnp.int32),
            pltpu.MemorySpace.VMEM((CHUNK,), jnp.float32),
            pltpu.MemorySpace.VMEM((CHUNK,), jnp.float32),
            pltpu.MemorySpace.VMEM((CHUNK,), jnp.float32),
            pltpu.MemorySpace.VMEM((CHUNK,), jnp.float32),
            pltpu.MemorySpace.VMEM((CHUNK,), jnp.float32),
            pltpu.MemorySpace.VMEM((CHUNK,), jnp.float32),
            pltpu.MemorySpace.VMEM((CHUNK,), jnp.float32),
            pltpu.MemorySpace.VMEM((CHUNK,), jnp.float32),
            pltpu.MemorySpace.VMEM((CHUNK,), jnp.float32),
            pltpu.MemorySpace.VMEM((8, LANES), jnp.float32),
            pltpu.MemorySpace.VMEM((LANES,), jnp.float32),
            pltpu.SemaphoreType.DMA,
            pltpu.SemaphoreType.DMA,
            pltpu.SemaphoreType.DMA,
            pltpu.SemaphoreType.DMA,
        ],
        compiler_params=pltpu.CompilerParams(needs_layout_passes=False),
    )
    return kfn(px, py, pz, nl, shx, shy, shz, params)


def kernel(positions, neigh_list, shifts, alpha, epsilon, r0, rcut1, rcut2):
    px = positions[:, 0]
    py = positions[:, 1]
    pz = positions[:, 2]
    shx = shifts[:, 0]
    shy = shifts[:, 1]
    shz = shifts[:, 2]
    inv = 1.0 / (rcut2 - rcut1)
    rows = [
        jnp.broadcast_to(alpha, (LANES,)),
        jnp.broadcast_to(alpha / r0, (LANES,)),
        jnp.broadcast_to(1.0 + rcut1 * inv, (LANES,)),
        jnp.broadcast_to(inv, (LANES,)),
        jnp.broadcast_to(0.5 * epsilon, (LANES,)),
        jnp.zeros((LANES,), jnp.float32),
        jnp.zeros((LANES,), jnp.float32),
        jnp.zeros((LANES,), jnp.float32),
    ]
    params = jnp.stack(rows).astype(jnp.float32)
    out = _run(px, py, pz, neigh_list, shx, shy, shz, params)
    energy = jnp.sum(out)
    return (energy,)
